# baseline (speedup 1.0000x reference)
import jax
import jax.numpy as jnp
from jax.experimental import pallas as pl
from jax.experimental.pallas import tpu as pltpu

_K = 32
_PASSES = 14
_R = 512
_NB = 8


def _fused_body(x_ref, w_ref, o_ref, f_scr, sq_scr, sqr_scr, c_scr):
    step = pl.program_id(0)

    @pl.when(step < _NB)
    def _matmul_phase():
        xb = x_ref[...].astype(jnp.bfloat16)
        wb = w_ref[...].astype(jnp.bfloat16)
        f = jnp.dot(xb, wb, preferred_element_type=jnp.float32)
        fb = f.astype(jnp.bfloat16)
        f_scr[pl.ds(step * _R, _R), :] = fb
        f32 = fb.astype(jnp.float32)
        sq = jnp.sum(f32 * f32, axis=1, keepdims=True)
        sq_scr[pl.ds(step * _R, _R), :] = sq
        sqr_scr[0:1, pl.ds(step * _R, _R)] = sq.reshape(1, _R)

    @pl.when(jnp.logical_and(step >= _NB, step < 2 * _NB))
    def _cross_phase():
        j = step - _NB
        fi = f_scr[pl.ds(j * _R, _R), :]
        cross = jax.lax.dot_general(
            fi, f_scr[...], (((1,), (1,)), ((), ())),
            preferred_element_type=jnp.float32)
        c_scr[pl.ds(jax.lax.rem(j, 2) * _R, _R), :] = cross

    @pl.when(step > _NB)
    def _select_phase():
        j = step - _NB - 1
        cross = c_scr[pl.ds(jax.lax.rem(j, 2) * _R, _R), :]
        sq_i = sq_scr[pl.ds(j * _R, _R), :]
        d2 = jnp.maximum(sq_i + sqr_scr[...] - 2.0 * cross, 1e-12)

        row_ids = j * _R + jax.lax.broadcasted_iota(jnp.int32, d2.shape, 0)
        col_ids = jax.lax.broadcasted_iota(jnp.int32, d2.shape, 1)

        x32 = jax.lax.bitcast_convert_type(d2, jnp.int32)
        x = jnp.where(row_ids == col_ids, jnp.int32(0x7FFFFFFF), x32)

        ans = jnp.zeros((d2.shape[0], 1), jnp.int32)
        for b in range(30, 30 - _PASSES, -1):
            t = ans | (1 << b)
            cnt = jnp.sum((x < t).astype(jnp.int32), axis=1, keepdims=True)
            ans = jnp.where(cnt <= _K, t, ans)
        ans = ans | (1 << (30 - _PASSES))

        o_ref[...] = jnp.sqrt(jax.lax.bitcast_convert_type(ans, jnp.float32))


def kernel(images, W):
    B, Din = images.shape
    D = W.shape[1]

    out = pl.pallas_call(
        _fused_body,
        grid=(2 * _NB + 1,),
        in_specs=[
            pl.BlockSpec((_R, Din), lambda i: (jnp.minimum(i, _NB - 1), 0)),
            pl.BlockSpec((Din, D), lambda i: (0, 0)),
        ],
        out_specs=pl.BlockSpec(
            (_R, 1), lambda i: (jnp.maximum(i - _NB - 1, 0), 0)),
        out_shape=jax.ShapeDtypeStruct((B, 1), jnp.float32),
        scratch_shapes=[
            pltpu.VMEM((B, D), jnp.bfloat16),
            pltpu.VMEM((B, 1), jnp.float32),
            pltpu.VMEM((1, B), jnp.float32),
            pltpu.VMEM((2 * _R, B), jnp.float32),
        ],
    )(images, W)

    return out.reshape(B)


# speedup vs baseline: 1.1189x; 1.1189x over previous
import jax
import jax.numpy as jnp
from jax.experimental import pallas as pl
from jax.experimental.pallas import tpu as pltpu

_K = 32
_PASSES = 14
_RM = 1024
_NM = 4
_R = 512
_NB = 8


def _fused_body(x_ref, w_ref, o_ref, f_scr, sq_scr, sqr_scr):
    step = pl.program_id(0)

    @pl.when(step < _NM)
    def _matmul_phase():
        xb = x_ref[...].astype(jnp.bfloat16)
        wb = w_ref[...].astype(jnp.bfloat16)
        f = jnp.dot(xb, wb, preferred_element_type=jnp.float32)
        fb = f.astype(jnp.bfloat16)
        f_scr[pl.ds(step * _RM, _RM), :] = fb
        f32 = fb.astype(jnp.float32)
        sq = jnp.sum(f32 * f32, axis=1, keepdims=True)
        sq_scr[pl.ds(step * _RM, _RM), :] = sq
        sqr_scr[0:1, pl.ds(step * _RM, _RM)] = sq.reshape(1, _RM)

    @pl.when(step >= _NM)
    def _select_phase():
        j = step - _NM
        fi = f_scr[pl.ds(j * _R, _R), :] * jnp.bfloat16(-2.0)
        cross2 = jax.lax.dot_general(
            fi, f_scr[...], (((1,), (1,)), ((), ())),
            preferred_element_type=jnp.float32)
        sq_i = sq_scr[pl.ds(j * _R, _R), :]
        d2 = jnp.maximum(sq_i + (sqr_scr[...] + cross2), 1e-12)

        x = jax.lax.bitcast_convert_type(d2, jnp.int32)

        ans = jnp.zeros((x.shape[0], 1), jnp.int32)
        for b in range(30, 30 - _PASSES, -1):
            t = ans | (1 << b)
            cnt = jnp.sum((x < t).astype(jnp.int32), axis=1, keepdims=True)
            ans = jnp.where(cnt <= _K + 1, t, ans)
        ans = ans | (1 << (30 - _PASSES))

        o_ref[...] = jnp.sqrt(jax.lax.bitcast_convert_type(ans, jnp.float32))


def kernel(images, W):
    B, Din = images.shape
    D = W.shape[1]

    out = pl.pallas_call(
        _fused_body,
        grid=(_NM + _NB,),
        in_specs=[
            pl.BlockSpec((_RM, Din), lambda i: (jnp.minimum(i, _NM - 1), 0)),
            pl.BlockSpec((Din, D), lambda i: (0, 0)),
        ],
        out_specs=pl.BlockSpec(
            (_R, 1), lambda i: (jnp.maximum(i - _NM, 0), 0)),
        out_shape=jax.ShapeDtypeStruct((B, 1), jnp.float32),
        scratch_shapes=[
            pltpu.VMEM((B, D), jnp.bfloat16),
            pltpu.VMEM((B, 1), jnp.float32),
            pltpu.VMEM((1, B), jnp.float32),
        ],
    )(images, W)

    return out.reshape(B)


# speedup vs baseline: 1.1694x; 1.0451x over previous
import jax
import jax.numpy as jnp
from jax.experimental import pallas as pl
from jax.experimental.pallas import tpu as pltpu

_K = 32
_PASSES = 13
_RM = 1024
_NM = 4
_R = 512
_NB = 8


def _fused_body(x_ref, w_ref, o_ref, f_scr, sq_scr, sqr_scr):
    step = pl.program_id(0)

    @pl.when(step < _NM)
    def _matmul_phase():
        xb = x_ref[...].astype(jnp.bfloat16)
        wb = w_ref[...].astype(jnp.bfloat16)
        f = jnp.dot(xb, wb, preferred_element_type=jnp.float32)
        fb = f.astype(jnp.bfloat16)
        f_scr[pl.ds(step * _RM, _RM), :] = fb
        f32 = fb.astype(jnp.float32)
        sq = jnp.sum(f32 * f32, axis=1, keepdims=True)
        sq_scr[pl.ds(step * _RM, _RM), :] = sq
        sqr_scr[0:1, pl.ds(step * _RM, _RM)] = sq.reshape(1, _RM)

    @pl.when(step >= _NM)
    def _select_phase():
        j = step - _NM
        fi = f_scr[pl.ds(j * _R, _R), :] * jnp.bfloat16(-2.0)
        cross2 = jax.lax.dot_general(
            fi, f_scr[...], (((1,), (1,)), ((), ())),
            preferred_element_type=jnp.float32)
        sq_i = sq_scr[pl.ds(j * _R, _R), :]
        d2 = jnp.maximum(sq_i + (sqr_scr[...] + cross2), 1e-12)

        x = jax.lax.bitcast_convert_type(d2, jnp.int32)

        ans = jnp.zeros((x.shape[0], 1), jnp.int32)
        for b in range(30, 30 - _PASSES, -1):
            t = ans | (1 << b)
            cnt = jnp.sum((x < t).astype(jnp.int32), axis=1, keepdims=True)
            ans = jnp.where(cnt <= _K + 1, t, ans)
        ans = ans | (1 << (30 - _PASSES))

        o_ref[...] = jnp.sqrt(jax.lax.bitcast_convert_type(ans, jnp.float32))


def kernel(images, W):
    B, Din = images.shape
    D = W.shape[1]

    out = pl.pallas_call(
        _fused_body,
        grid=(_NM + _NB,),
        in_specs=[
            pl.BlockSpec((_RM, Din), lambda i: (jnp.minimum(i, _NM - 1), 0)),
            pl.BlockSpec((Din, D), lambda i: (0, 0)),
        ],
        out_specs=pl.BlockSpec(
            (_R, 1), lambda i: (jnp.maximum(i - _NM, 0), 0)),
        out_shape=jax.ShapeDtypeStruct((B, 1), jnp.float32),
        scratch_shapes=[
            pltpu.VMEM((B, D), jnp.bfloat16),
            pltpu.VMEM((B, 1), jnp.float32),
            pltpu.VMEM((1, B), jnp.float32),
        ],
    )(images, W)

    return out.reshape(B)


# speedup vs baseline: 1.1738x; 1.0038x over previous
import jax
import jax.numpy as jnp
from jax.experimental import pallas as pl
from jax.experimental.pallas import tpu as pltpu

_K = 32
_PASSES = 13
_RM = 1024
_NM = 4
_R = 1024
_NB = 4


def _fused_body(x_ref, w_ref, o_ref, f_scr, sq_scr, sqr_scr):
    step = pl.program_id(0)

    @pl.when(step < _NM)
    def _matmul_phase():
        xb = x_ref[...].astype(jnp.bfloat16)
        wb = w_ref[...].astype(jnp.bfloat16)
        f = jnp.dot(xb, wb, preferred_element_type=jnp.float32)
        fb = f.astype(jnp.bfloat16)
        f_scr[pl.ds(step * _RM, _RM), :] = fb
        f32 = fb.astype(jnp.float32)
        sq = jnp.sum(f32 * f32, axis=1, keepdims=True)
        sq_scr[pl.ds(step * _RM, _RM), :] = sq
        sqr_scr[0:1, pl.ds(step * _RM, _RM)] = sq.reshape(1, _RM)

    @pl.when(step >= _NM)
    def _select_phase():
        j = step - _NM
        fi = f_scr[pl.ds(j * _R, _R), :] * jnp.bfloat16(-2.0)
        cross2 = jax.lax.dot_general(
            fi, f_scr[...], (((1,), (1,)), ((), ())),
            preferred_element_type=jnp.float32)
        sq_i = sq_scr[pl.ds(j * _R, _R), :]
        d2 = jnp.maximum(sq_i + (sqr_scr[...] + cross2), 1e-12)

        x = jax.lax.bitcast_convert_type(d2, jnp.int32)

        ans = jnp.zeros((x.shape[0], 1), jnp.int32)
        for b in range(30, 30 - _PASSES, -1):
            t = ans | (1 << b)
            cnt = jnp.sum((x < t).astype(jnp.int32), axis=1, keepdims=True)
            ans = jnp.where(cnt <= _K + 1, t, ans)
        ans = ans | (1 << (30 - _PASSES))

        o_ref[...] = jnp.sqrt(jax.lax.bitcast_convert_type(ans, jnp.float32))


def kernel(images, W):
    B, Din = images.shape
    D = W.shape[1]

    out = pl.pallas_call(
        _fused_body,
        grid=(_NM + _NB,),
        in_specs=[
            pl.BlockSpec((_RM, Din), lambda i: (jnp.minimum(i, _NM - 1), 0)),
            pl.BlockSpec((Din, D), lambda i: (0, 0)),
        ],
        out_specs=pl.BlockSpec(
            (_R, 1), lambda i: (jnp.maximum(i - _NM, 0), 0)),
        out_shape=jax.ShapeDtypeStruct((B, 1), jnp.float32),
        scratch_shapes=[
            pltpu.VMEM((B, D), jnp.bfloat16),
            pltpu.VMEM((B, 1), jnp.float32),
            pltpu.VMEM((1, B), jnp.float32),
        ],
    )(images, W)

    return out.reshape(B)


# speedup vs baseline: 1.2101x; 1.0309x over previous
import jax
import jax.numpy as jnp
from jax.experimental import pallas as pl
from jax.experimental.pallas import tpu as pltpu

_K = 32
_PASSES = 13
_RM = 1024
_NM = 4
_R = 1024
_NB = 4


def _fused_body(x_ref, w_ref, o_ref, f_scr, sq_scr, sqr_scr):
    step = pl.program_id(0)

    @pl.when(step < _NM)
    def _matmul_phase():
        xb = x_ref[...].astype(jnp.bfloat16)
        wb = w_ref[...].astype(jnp.bfloat16)
        f = jnp.dot(xb, wb, preferred_element_type=jnp.float32)
        fb = f.astype(jnp.bfloat16)
        f_scr[pl.ds(step * _RM, _RM), :] = fb
        f32 = fb.astype(jnp.float32)
        sq = jnp.sum(f32 * f32, axis=1, keepdims=True)
        sq_scr[pl.ds(step * _RM, _RM), :] = sq
        sqr_scr[0:1, pl.ds(step * _RM, _RM)] = sq.reshape(1, _RM)

    @pl.when(step >= _NM)
    def _select_phase():
        j = step - _NM
        fi = f_scr[pl.ds(j * _R, _R), :] * jnp.bfloat16(-2.0)
        cross2 = jax.lax.dot_general(
            fi, f_scr[...], (((1,), (1,)), ((), ())),
            preferred_element_type=jnp.float32)
        sq_i = sq_scr[pl.ds(j * _R, _R), :]
        d2 = jnp.maximum(sq_i + (sqr_scr[...] + cross2), 1e-12)

        x = jax.lax.bitcast_convert_type(d2, jnp.int32)
        n = x.shape[1] // 2
        hi = x[:, :n] & jnp.int32(-0x10000)
        lo = jax.lax.shift_right_logical(x[:, n:], 16)
        xp = (hi | lo) | jnp.int32(-0x7FFF8000)

        ge_needed = 2 * n - (_K + 1)
        ans = jnp.zeros((x.shape[0], 1), jnp.int32)
        for b in range(14, -1, -1):
            t = ans | (1 << b)
            tp = t * jnp.int32(0x10001)
            ge = (xp - tp) & jnp.int32(-0x7FFF8000)
            contrib = jax.lax.shift_right_logical(ge, 15)
            s = jnp.sum(contrib, axis=1, keepdims=True)
            cnt_ge = (s & 0xFFFF) + jax.lax.shift_right_logical(s, 16)
            ans = jnp.where(cnt_ge >= ge_needed, t, ans)

        full = (ans << 16) | (1 << 15)
        o_ref[...] = jnp.sqrt(jax.lax.bitcast_convert_type(full, jnp.float32))


def kernel(images, W):
    B, Din = images.shape
    D = W.shape[1]

    out = pl.pallas_call(
        _fused_body,
        grid=(_NM + _NB,),
        in_specs=[
            pl.BlockSpec((_RM, Din), lambda i: (jnp.minimum(i, _NM - 1), 0)),
            pl.BlockSpec((Din, D), lambda i: (0, 0)),
        ],
        out_specs=pl.BlockSpec(
            (_R, 1), lambda i: (jnp.maximum(i - _NM, 0), 0)),
        out_shape=jax.ShapeDtypeStruct((B, 1), jnp.float32),
        scratch_shapes=[
            pltpu.VMEM((B, D), jnp.bfloat16),
            pltpu.VMEM((B, 1), jnp.float32),
            pltpu.VMEM((1, B), jnp.float32),
        ],
    )(images, W)

    return out.reshape(B)
